# baseline (speedup 1.0000x reference)
import jax
import jax.numpy as jnp
from jax.experimental import pallas as pl
from jax.experimental.pallas import tpu as pltpu

_BS0 = 1024
_SLOTS = 8
_SUB = 8
_LANE = 128


def _make_kernel(total_fetches, bs0, nk, s1):
    lookahead = _SLOTS // nk - 1

    def body(idx_sref, probs_sref, pv_ref, out_ref, buf_ref, sem_ref):
        i = pl.program_id(0)
        b = pl.program_id(1)
        nb = pl.num_programs(1)
        flat = i * nb + b

        def copy_for(g, slot):
            i_g = g // (nb * nk)
            c = idx_sref[g % (nb * nk)]
            src = pv_ref.at[pl.ds(i_g * bs0, bs0), c // _SUB, :, c % _SUB, :]
            return pltpu.make_async_copy(src, buf_ref.at[slot],
                                         sem_ref.at[slot])

        @pl.when(flat == 0)
        def _first():
            for g in range(min(lookahead * nk, total_fetches)):
                copy_for(g, g % _SLOTS).start()

        g_next = (flat + lookahead) * nk
        for kk in range(nk):
            @pl.when(g_next + kk < total_fetches)
            def _prefetch(kk=kk):
                copy_for(g_next + kk, (g_next + kk) % _SLOTS).start()

        acc = None
        for kk in range(nk):
            g = flat * nk + kk
            copy_for(g, g % _SLOTS).wait()
            v = buf_ref[g % _SLOTS]
            p = probs_sref[b * nk + kk]
            acc = p * v if acc is None else acc + p * v
        out_ref[0] = acc.reshape(bs0, s1)

    return body


def kernel(selection_probabilities, parameter, selection_index):
    s0, s1, bank = parameter.shape
    b, k = selection_index.shape
    cb, jb = bank // _SUB, s1 // _LANE
    pv = jnp.transpose(parameter, (0, 2, 1))
    pv = pv.reshape(s0, cb, _SUB, jb, _LANE)
    pv = jnp.transpose(pv, (0, 1, 3, 2, 4))
    idx_flat = selection_index.reshape(-1)
    probs_flat = selection_probabilities.reshape(-1)
    grid = (s0 // _BS0, b)
    total_fetches = (s0 // _BS0) * b * k
    out = pl.pallas_call(
        _make_kernel(total_fetches, _BS0, k, s1),
        grid_spec=pltpu.PrefetchScalarGridSpec(
            num_scalar_prefetch=2,
            grid=grid,
            in_specs=[pl.BlockSpec(memory_space=pl.ANY)],
            out_specs=pl.BlockSpec(
                (1, _BS0, s1),
                lambda i, bb, idx, pr: (bb, i, 0)),
            scratch_shapes=[
                pltpu.VMEM((_SLOTS, _BS0, jb, _LANE), jnp.float32),
                pltpu.SemaphoreType.DMA((_SLOTS,)),
            ],
        ),
        out_shape=jax.ShapeDtypeStruct((b, s0, s1), jnp.float32),
    )(idx_flat, probs_flat, pv)
    return out


# speedup vs baseline: 1.0077x; 1.0077x over previous
import jax
import jax.numpy as jnp
from jax.experimental import pallas as pl
from jax.experimental.pallas import tpu as pltpu

_BS0 = 512
_SLOTS = 16
_SUB = 8
_LANE = 128


def _make_kernel(total_fetches, bs0, nk, s1):
    lookahead = _SLOTS // nk - 1

    def body(idx_sref, probs_sref, pv_ref, out_ref, buf_ref, sem_ref):
        i = pl.program_id(0)
        b = pl.program_id(1)
        nb = pl.num_programs(1)
        flat = i * nb + b

        def copy_for(g, slot):
            i_g = g // (nb * nk)
            c = idx_sref[g % (nb * nk)]
            src = pv_ref.at[pl.ds(i_g * bs0, bs0), c // _SUB, :, c % _SUB, :]
            return pltpu.make_async_copy(src, buf_ref.at[slot],
                                         sem_ref.at[slot])

        @pl.when(flat == 0)
        def _first():
            for g in range(min(lookahead * nk, total_fetches)):
                copy_for(g, g % _SLOTS).start()

        g_next = (flat + lookahead) * nk
        for kk in range(nk):
            @pl.when(g_next + kk < total_fetches)
            def _prefetch(kk=kk):
                copy_for(g_next + kk, (g_next + kk) % _SLOTS).start()

        acc = None
        for kk in range(nk):
            g = flat * nk + kk
            copy_for(g, g % _SLOTS).wait()
            v = buf_ref[g % _SLOTS]
            p = probs_sref[b * nk + kk]
            acc = p * v if acc is None else acc + p * v
        out_ref[0] = acc.reshape(bs0, s1)

    return body


def kernel(selection_probabilities, parameter, selection_index):
    s0, s1, bank = parameter.shape
    b, k = selection_index.shape
    cb, jb = bank // _SUB, s1 // _LANE
    pv = jnp.transpose(parameter, (0, 2, 1))
    pv = pv.reshape(s0, cb, _SUB, jb, _LANE)
    pv = jnp.transpose(pv, (0, 1, 3, 2, 4))
    idx_flat = selection_index.reshape(-1)
    probs_flat = selection_probabilities.reshape(-1)
    grid = (s0 // _BS0, b)
    total_fetches = (s0 // _BS0) * b * k
    out = pl.pallas_call(
        _make_kernel(total_fetches, _BS0, k, s1),
        grid_spec=pltpu.PrefetchScalarGridSpec(
            num_scalar_prefetch=2,
            grid=grid,
            in_specs=[pl.BlockSpec(memory_space=pl.ANY)],
            out_specs=pl.BlockSpec(
                (1, _BS0, s1),
                lambda i, bb, idx, pr: (bb, i, 0)),
            scratch_shapes=[
                pltpu.VMEM((_SLOTS, _BS0, jb, _LANE), jnp.float32),
                pltpu.SemaphoreType.DMA((_SLOTS,)),
            ],
        ),
        out_shape=jax.ShapeDtypeStruct((b, s0, s1), jnp.float32),
    )(idx_flat, probs_flat, pv)
    return out
